# baseline (speedup 1.0000x reference)
import functools

import jax
import jax.numpy as jnp
from jax import lax
from jax.experimental import pallas as pl
from jax.experimental.pallas import tpu as pltpu
from jax.experimental.pallas import tpu_sc as plsc

CHUNK = 128


def _sc_gather4(uid2, iid2, P, Q, U, V):
    n_chunks = uid2.shape[0]
    B = n_chunks * CHUNK
    D = P.shape[1]
    info = plsc.get_sparse_core_info()
    NC, NS = info.num_cores, info.num_subcores
    NW = NC * NS
    chunks_per_w = n_chunks // NW

    mesh = plsc.VectorSubcoreMesh(core_axis_name="c", subcore_axis_name="s")

    @functools.partial(
        pl.kernel,
        mesh=mesh,
        compiler_params=pltpu.CompilerParams(use_tc_tiling_on_sc=False),
        out_type=[jax.ShapeDtypeStruct((B, D), jnp.float32)] * 4,
        scratch_types=[
            pltpu.VMEM((CHUNK,), jnp.int32),
            pltpu.VMEM((CHUNK,), jnp.int32),
            pltpu.VMEM((CHUNK, D), jnp.float32),
            pltpu.VMEM((CHUNK, D), jnp.float32),
            pltpu.VMEM((CHUNK, D), jnp.float32),
            pltpu.VMEM((CHUNK, D), jnp.float32),
            pltpu.SemaphoreType.DMA,
            pltpu.SemaphoreType.DMA,
            pltpu.SemaphoreType.DMA,
            pltpu.SemaphoreType.DMA,
        ],
    )
    def gather_kernel(uid_hbm, iid_hbm, p_hbm, q_hbm, u_hbm, v_hbm,
                      p_out, q_out, u_out, v_out,
                      uidx, iidx, bp, bq, bu, bv, s0, s1, s2, s3):
        wid = lax.axis_index("s") * NC + lax.axis_index("c")
        for j in range(chunks_per_w):
            r = wid * chunks_per_w + j
            base = r * CHUNK
            pltpu.sync_copy(uid_hbm.at[r], uidx)
            pltpu.sync_copy(iid_hbm.at[r], iidx)
            cp = pltpu.async_copy(p_hbm.at[uidx], bp, s0)
            cq = pltpu.async_copy(q_hbm.at[iidx], bq, s1)
            cu = pltpu.async_copy(u_hbm.at[uidx], bu, s2)
            cv = pltpu.async_copy(v_hbm.at[iidx], bv, s3)
            cp.wait()
            pltpu.sync_copy(bp, p_out.at[pl.ds(base, CHUNK)])
            cq.wait()
            pltpu.sync_copy(bq, q_out.at[pl.ds(base, CHUNK)])
            cu.wait()
            pltpu.sync_copy(bu, u_out.at[pl.ds(base, CHUNK)])
            cv.wait()
            pltpu.sync_copy(bv, v_out.at[pl.ds(base, CHUNK)])

    return gather_kernel(uid2, iid2, P, Q, U, V)


def _head_body(p_ref, q_ref, u_ref, v_ref, w1_ref, b1_ref, w2_ref, b2_ref,
               wp_ref, bp_ref, out_ref):
    gmf = p_ref[...] * q_ref[...]
    x = jnp.concatenate([u_ref[...], v_ref[...]], axis=1)
    h = lax.dot_general(x, w1_ref[...], (((1,), (1,)), ((), ())),
                        preferred_element_type=jnp.float32)
    h = jnp.maximum(h + b1_ref[...], 0.0)
    mlp = lax.dot_general(h, w2_ref[...], (((1,), (1,)), ((), ())),
                          preferred_element_type=jnp.float32)
    mlp = jnp.maximum(mlp + b2_ref[...], 0.0)
    con = jnp.concatenate([gmf, mlp], axis=1)
    z = jnp.sum(con * wp_ref[...], axis=1, keepdims=True) + bp_ref[0]
    out_ref[...] = 1.0 / (1.0 + jnp.exp(-z))


def _tc_head(p_mf, q_mf, u_mlp, v_mlp, W1, b1, W2, b2, Wp, bp, interpret=False):
    B, D = p_mf.shape
    H = W1.shape[0]
    BLK = 2048
    grid = (B // BLK,)
    row_spec = pl.BlockSpec((BLK, D), lambda i: (i, 0))
    full = lambda shape: pl.BlockSpec(shape, lambda i: (0, 0))
    return pl.pallas_call(
        _head_body,
        grid=grid,
        in_specs=[
            row_spec, row_spec, row_spec, row_spec,
            full(W1.shape), full((1, H)),
            full(W2.shape), full((1, D)),
            full(Wp.shape), pl.BlockSpec(memory_space=pltpu.SMEM),
        ],
        out_specs=pl.BlockSpec((BLK, 1), lambda i: (i, 0)),
        out_shape=jax.ShapeDtypeStruct((B, 1), jnp.float32),
        compiler_params=pltpu.CompilerParams(
            dimension_semantics=("arbitrary",)),
        interpret=interpret,
    )(p_mf, q_mf, u_mlp, v_mlp, W1, b1.reshape(1, H), W2, b2.reshape(1, D),
      Wp, bp)


def kernel(user_id, item_id, P, Q, U, V, W1, b1, W2, b2, Wp, bp):
    B = user_id.shape[0]
    uid2 = user_id.astype(jnp.int32).reshape(B // CHUNK, CHUNK)
    iid2 = item_id.astype(jnp.int32).reshape(B // CHUNK, CHUNK)
    p_mf, q_mf, u_mlp, v_mlp = _sc_gather4(uid2, iid2, P, Q, U, V)
    return _tc_head(p_mf, q_mf, u_mlp, v_mlp, W1, b1, W2, b2, Wp, bp)


# speedup vs baseline: 1.1742x; 1.1742x over previous
import functools

import jax
import jax.numpy as jnp
from jax import lax
from jax.experimental import pallas as pl
from jax.experimental.pallas import tpu as pltpu
from jax.experimental.pallas import tpu_sc as plsc


def _sc_gather_pairs(uid, iid, PU, QV):
    B = uid.shape[0]
    W = PU.shape[1]
    info = plsc.get_sparse_core_info()
    NC, NS = info.num_cores, info.num_subcores
    NW = NC * NS
    b_per_w = B // NW

    mesh = plsc.VectorSubcoreMesh(core_axis_name="c", subcore_axis_name="s")

    @functools.partial(
        pl.kernel,
        mesh=mesh,
        out_type=[jax.ShapeDtypeStruct((B, W), jnp.float32)] * 2,
        scratch_types=[
            pltpu.VMEM((b_per_w,), jnp.int32),
            pltpu.VMEM((b_per_w,), jnp.int32),
            pltpu.VMEM((b_per_w, W), jnp.float32),
            pltpu.SemaphoreType.DMA,
        ],
    )
    def gather_kernel(uid_hbm, iid_hbm, pu_hbm, qv_hbm,
                      out_u, out_i,
                      uidx, iidx, rows_v, sem):
        wid = lax.axis_index("s") * NC + lax.axis_index("c")
        base = wid * b_per_w
        pltpu.sync_copy(uid_hbm.at[pl.ds(base, b_per_w)], uidx)
        pltpu.sync_copy(iid_hbm.at[pl.ds(base, b_per_w)], iidx)
        pltpu.async_copy(pu_hbm.at[uidx], rows_v, sem).wait()
        pltpu.sync_copy(rows_v, out_u.at[pl.ds(base, b_per_w)])
        pltpu.async_copy(qv_hbm.at[iidx], rows_v, sem).wait()
        pltpu.sync_copy(rows_v, out_i.at[pl.ds(base, b_per_w)])

    return gather_kernel(uid, iid, PU, QV)


def _head_body(pu_ref, qv_ref, w1_ref, b1_ref, w2_ref, b2_ref,
               wp_ref, bp_ref, out_ref):
    pu = pu_ref[...]
    qv = qv_ref[...]
    D = pu.shape[1] // 2
    gmf = pu[:, :D] * qv[:, :D]
    x = jnp.concatenate([pu[:, D:], qv[:, D:]], axis=1)
    h = lax.dot_general(x, w1_ref[...], (((1,), (1,)), ((), ())),
                        preferred_element_type=jnp.float32)
    h = jnp.maximum(h + b1_ref[...], 0.0)
    mlp = lax.dot_general(h, w2_ref[...], (((1,), (1,)), ((), ())),
                          preferred_element_type=jnp.float32)
    mlp = jnp.maximum(mlp + b2_ref[...], 0.0)
    con = jnp.concatenate([gmf, mlp], axis=1)
    z = jnp.sum(con * wp_ref[...], axis=1, keepdims=True)
    out_ref[...] = 1.0 / (1.0 + jnp.exp(-(z + bp_ref[0, 0])))


def _tc_head(rows_u, rows_i, W1, b1, W2, b2, Wp, bp, interpret=False):
    B, W = rows_u.shape
    BLK = 2048
    grid = (B // BLK,)
    blk_spec = pl.BlockSpec((BLK, W), lambda i: (i, 0))
    full = lambda shape: pl.BlockSpec(shape, lambda i: (0, 0))
    return pl.pallas_call(
        _head_body,
        grid=grid,
        in_specs=[
            blk_spec, blk_spec,
            full(W1.shape), full(b1.shape),
            full(W2.shape), full(b2.shape),
            full(Wp.shape),
            pl.BlockSpec(memory_space=pltpu.SMEM),
        ],
        out_specs=pl.BlockSpec((BLK, 1), lambda i: (i, 0)),
        out_shape=jax.ShapeDtypeStruct((B, 1), jnp.float32),
        compiler_params=pltpu.CompilerParams(
            dimension_semantics=("arbitrary",)),
        interpret=interpret,
    )(rows_u, rows_i, W1, b1, W2, b2, Wp, bp)


def kernel(user_id, item_id, P, Q, U, V, W1, b1, W2, b2, Wp, bp):
    uid = user_id.astype(jnp.int32)
    iid = item_id.astype(jnp.int32)
    D = P.shape[1]
    PU = jnp.pad(P, ((0, 0), (0, D))) + jnp.pad(U, ((0, 0), (D, 0)))
    QV = jnp.pad(Q, ((0, 0), (0, D))) + jnp.pad(V, ((0, 0), (D, 0)))
    rows_u, rows_i = _sc_gather_pairs(uid, iid, PU, QV)
    return _tc_head(rows_u, rows_i,
                    W1, b1[None, :], W2, b2[None, :], Wp, bp[None, :])


# speedup vs baseline: 2.0428x; 1.7397x over previous
import functools

import jax
import jax.numpy as jnp
from jax import lax
from jax.experimental import pallas as pl
from jax.experimental.pallas import tpu as pltpu
from jax.experimental.pallas import tpu_sc as plsc

WIN = 128


def _sc_stream_gather(su, ou, si, oi, PT, UT, QT, VT, TP, TU, TQ, TV):
    B = su.shape[0]
    Df = PT.shape[0]
    V = PT.shape[1]
    w_last = (V - 2) // WIN
    info = plsc.get_sparse_core_info()
    NC, NS = info.num_cores, info.num_subcores
    NW = NC * NS
    b_per_w = B // NW
    W2 = 2 * Df

    mesh = plsc.VectorSubcoreMesh(core_axis_name="c", subcore_axis_name="s")

    @functools.partial(
        pl.kernel,
        mesh=mesh,
        compiler_params=pltpu.CompilerParams(needs_layout_passes=False),
        out_type=[jax.ShapeDtypeStruct((B, W2), jnp.float32)] * 2,
        scratch_types=[
            pltpu.VMEM((b_per_w + 16,), jnp.int32),
            pltpu.VMEM((b_per_w,), jnp.int32),
            pltpu.VMEM((Df, WIN), jnp.float32),
            pltpu.VMEM((Df, WIN), jnp.float32),
            pltpu.VMEM((b_per_w, W2), jnp.float32),
        ],
    )
    def gather_kernel(su_hbm, ou_hbm, si_hbm, oi_hbm, pt, ut, qt, vt,
                      tp, tu, tq, tv,
                      out_u, out_i,
                      vs_vm, dest_v, blk_a, blk_b, rows):
        wid = lax.axis_index("s") * NC + lax.axis_index("c")
        base = wid * b_per_w

        def run_stream(v_hbm, o_hbm, ta, tb, tail_a, tail_b, out):
            pltpu.sync_copy(v_hbm.at[pl.ds(base, b_per_w)],
                            vs_vm.at[pl.ds(0, b_per_w)])

            def body(j, cur_win):
                v = jnp.minimum(vs_vm[pl.ds(j, 16)][0], V - 2)
                win = lax.div(v, WIN)
                changed = jnp.not_equal(win, cur_win)

                @pl.when(jnp.logical_and(changed, win < w_last))
                def _():
                    off = pl.multiple_of(win * WIN, WIN)
                    pltpu.sync_copy(ta.at[:, pl.ds(off, WIN)], blk_a)
                    pltpu.sync_copy(tb.at[:, pl.ds(off, WIN)], blk_b)

                @pl.when(jnp.logical_and(changed, win >= w_last))
                def _():
                    pltpu.sync_copy(tail_a, blk_a)
                    pltpu.sync_copy(tail_b, blk_b)

                c = jnp.full((16,), lax.rem(v, WIN), dtype=jnp.int32)
                for k in range(Df // 16):
                    r = jnp.arange(16 * k, 16 * (k + 1), dtype=jnp.int32)
                    rows[j, pl.ds(16 * k, 16)] = plsc.load_gather(blk_a, [r, c])
                    rows[j, pl.ds(Df + 16 * k, 16)] = plsc.load_gather(
                        blk_b, [r, c])
                return win

            lax.fori_loop(0, b_per_w, body, jnp.int32(-1))

            pltpu.sync_copy(o_hbm.at[pl.ds(base, b_per_w)], dest_v)
            for k in range(b_per_w // 128):
                pltpu.sync_copy(
                    rows.at[pl.ds(128 * k, 128)],
                    out.at[dest_v.at[pl.ds(128 * k, 128)]])

        run_stream(su_hbm, ou_hbm, pt, ut, tp, tu, out_u)
        run_stream(si_hbm, oi_hbm, qt, vt, tq, tv, out_i)

    return gather_kernel(su, ou, si, oi, PT, UT, QT, VT, TP, TU, TQ, TV)


def _head_body(pu_ref, qv_ref, w1_ref, b1_ref, w2_ref, b2_ref,
               wp_ref, bp_ref, out_ref):
    pu = pu_ref[...]
    qv = qv_ref[...]
    D = pu.shape[1] // 2
    gmf = pu[:, :D] * qv[:, :D]
    x = jnp.concatenate([pu[:, D:], qv[:, D:]], axis=1)
    h = lax.dot_general(x, w1_ref[...], (((1,), (1,)), ((), ())),
                        preferred_element_type=jnp.float32)
    h = jnp.maximum(h + b1_ref[...], 0.0)
    mlp = lax.dot_general(h, w2_ref[...], (((1,), (1,)), ((), ())),
                          preferred_element_type=jnp.float32)
    mlp = jnp.maximum(mlp + b2_ref[...], 0.0)
    con = jnp.concatenate([gmf, mlp], axis=1)
    z = jnp.sum(con * wp_ref[...], axis=1, keepdims=True)
    out_ref[...] = 1.0 / (1.0 + jnp.exp(-(z + bp_ref[0, 0])))


def _tc_head(rows_u, rows_i, W1, b1, W2, b2, Wp, bp, interpret=False):
    B, W = rows_u.shape
    BLK = 2048
    grid = (B // BLK,)
    blk_spec = pl.BlockSpec((BLK, W), lambda i: (i, 0))
    full = lambda shape: pl.BlockSpec(shape, lambda i: (0, 0))
    return pl.pallas_call(
        _head_body,
        grid=grid,
        in_specs=[
            blk_spec, blk_spec,
            full(W1.shape), full(b1.shape),
            full(W2.shape), full(b2.shape),
            full(Wp.shape),
            pl.BlockSpec(memory_space=pltpu.SMEM),
        ],
        out_specs=pl.BlockSpec((BLK, 1), lambda i: (i, 0)),
        out_shape=jax.ShapeDtypeStruct((B, 1), jnp.float32),
        compiler_params=pltpu.CompilerParams(
            dimension_semantics=("arbitrary",)),
        interpret=interpret,
    )(rows_u, rows_i, W1, b1, W2, b2, Wp, bp)


def kernel(user_id, item_id, P, Q, U, V, W1, b1, W2, b2, Wp, bp):
    B = user_id.shape[0]
    uid = user_id.astype(jnp.int32)
    iid = item_id.astype(jnp.int32)
    pos = jnp.arange(B, dtype=jnp.int32)
    su, ou = lax.sort_key_val(uid, pos)
    si, oi = lax.sort_key_val(iid, pos)
    V0 = P.shape[0]
    t0 = ((V0 - 2) // WIN) * WIN
    tpad = ((0, 0), (0, WIN - (V0 - t0)))
    TP, TU, TQ, TV = (jnp.pad(T[:, t0:], tpad) for T in (P.T, U.T, Q.T, V.T))
    rows_u, rows_i = _sc_stream_gather(su, ou, si, oi, P.T, U.T, Q.T, V.T,
                                       TP, TU, TQ, TV)
    return _tc_head(rows_u, rows_i,
                    W1, b1[None, :], W2, b2[None, :], Wp, bp[None, :])


# speedup vs baseline: 3.0905x; 1.5128x over previous
import functools

import jax
import jax.numpy as jnp
from jax import lax
from jax.experimental import pallas as pl
from jax.experimental.pallas import tpu as pltpu
from jax.experimental.pallas import tpu_sc as plsc

WIN = 256
TAILW = 128


def _sc_stream_gather(su, ou, si, oi, PT, UT, QT, VT, TP, TU, TQ, TV):
    B = su.shape[0]
    Df = PT.shape[0]
    V = PT.shape[1]
    w_last = (V - 2) // WIN
    info = plsc.get_sparse_core_info()
    NC, NS = info.num_cores, info.num_subcores
    NW = NC * NS
    b_per_w = B // NW
    W2 = 2 * Df

    mesh = plsc.VectorSubcoreMesh(core_axis_name="c", subcore_axis_name="s")

    @functools.partial(
        pl.kernel,
        mesh=mesh,
        compiler_params=pltpu.CompilerParams(needs_layout_passes=False),
        out_type=[jax.ShapeDtypeStruct((B, W2), jnp.float32)] * 2,
        scratch_types=[
            pltpu.VMEM((b_per_w + 16,), jnp.int32),
            pltpu.VMEM((b_per_w,), jnp.int32),
            pltpu.VMEM((Df, WIN), jnp.float32),
            pltpu.VMEM((Df, WIN), jnp.float32),
            pltpu.VMEM((b_per_w, W2), jnp.float32),
            pltpu.SemaphoreType.DMA,
            pltpu.SemaphoreType.DMA,
        ],
    )
    def gather_kernel(su_hbm, ou_hbm, si_hbm, oi_hbm, pt, ut, qt, vt,
                      tp, tu, tq, tv,
                      out_u, out_i,
                      vs_vm, dest_v, blk_a, blk_b, rows, sa, sb):
        wid = lax.axis_index("s") * NC + lax.axis_index("c")
        base = wid * b_per_w

        def run_stream(v_hbm, o_hbm, ta, tb, tail_a, tail_b, out):
            pltpu.sync_copy(v_hbm.at[pl.ds(base, b_per_w)],
                            vs_vm.at[pl.ds(0, b_per_w)])

            def body(j, cur_win):
                v = jnp.minimum(vs_vm[pl.ds(j, 16)][0], V - 2)
                win = lax.div(v, WIN)
                changed = jnp.not_equal(win, cur_win)

                @pl.when(jnp.logical_and(changed, win < w_last))
                def _():
                    off = pl.multiple_of(win * WIN, WIN)
                    ca = pltpu.async_copy(ta.at[:, pl.ds(off, WIN)], blk_a, sa)
                    cb = pltpu.async_copy(tb.at[:, pl.ds(off, WIN)], blk_b, sb)
                    ca.wait()
                    cb.wait()

                @pl.when(jnp.logical_and(changed, win >= w_last))
                def _():
                    ca = pltpu.async_copy(tail_a,
                                          blk_a.at[:, pl.ds(0, TAILW)], sa)
                    cb = pltpu.async_copy(tail_b,
                                          blk_b.at[:, pl.ds(0, TAILW)], sb)
                    ca.wait()
                    cb.wait()

                c = jnp.full((16,), lax.rem(v, WIN), dtype=jnp.int32)
                for k in range(Df // 16):
                    r = jnp.arange(16 * k, 16 * (k + 1), dtype=jnp.int32)
                    rows[j, pl.ds(16 * k, 16)] = plsc.load_gather(blk_a, [r, c])
                    rows[j, pl.ds(Df + 16 * k, 16)] = plsc.load_gather(
                        blk_b, [r, c])
                return win

            lax.fori_loop(0, b_per_w, body, jnp.int32(-1))

            pltpu.sync_copy(o_hbm.at[pl.ds(base, b_per_w)], dest_v)
            for k in range(b_per_w // 128):
                pltpu.sync_copy(
                    rows.at[pl.ds(128 * k, 128)],
                    out.at[dest_v.at[pl.ds(128 * k, 128)]])

        run_stream(su_hbm, ou_hbm, pt, ut, tp, tu, out_u)
        run_stream(si_hbm, oi_hbm, qt, vt, tq, tv, out_i)

    return gather_kernel(su, ou, si, oi, PT, UT, QT, VT, TP, TU, TQ, TV)


def _head_body(pu_ref, qv_ref, w1_ref, b1_ref, w2_ref, b2_ref,
               wp_ref, bp_ref, out_ref):
    pu = pu_ref[...]
    qv = qv_ref[...]
    D = pu.shape[1] // 2
    gmf = pu[:, :D] * qv[:, :D]
    x = jnp.concatenate([pu[:, D:], qv[:, D:]], axis=1)
    h = lax.dot_general(x, w1_ref[...], (((1,), (1,)), ((), ())),
                        preferred_element_type=jnp.float32)
    h = jnp.maximum(h + b1_ref[...], 0.0)
    mlp = lax.dot_general(h, w2_ref[...], (((1,), (1,)), ((), ())),
                          preferred_element_type=jnp.float32)
    mlp = jnp.maximum(mlp + b2_ref[...], 0.0)
    con = jnp.concatenate([gmf, mlp], axis=1)
    z = jnp.sum(con * wp_ref[...], axis=1, keepdims=True)
    out_ref[...] = 1.0 / (1.0 + jnp.exp(-(z + bp_ref[0, 0])))


def _tc_head(rows_u, rows_i, W1, b1, W2, b2, Wp, bp, interpret=False):
    B, W = rows_u.shape
    BLK = 2048
    grid = (B // BLK,)
    blk_spec = pl.BlockSpec((BLK, W), lambda i: (i, 0))
    full = lambda shape: pl.BlockSpec(shape, lambda i: (0, 0))
    return pl.pallas_call(
        _head_body,
        grid=grid,
        in_specs=[
            blk_spec, blk_spec,
            full(W1.shape), full(b1.shape),
            full(W2.shape), full(b2.shape),
            full(Wp.shape),
            pl.BlockSpec(memory_space=pltpu.SMEM),
        ],
        out_specs=pl.BlockSpec((BLK, 1), lambda i: (i, 0)),
        out_shape=jax.ShapeDtypeStruct((B, 1), jnp.float32),
        compiler_params=pltpu.CompilerParams(
            dimension_semantics=("arbitrary",)),
        interpret=interpret,
    )(rows_u, rows_i, W1, b1, W2, b2, Wp, bp)


def kernel(user_id, item_id, P, Q, U, V, W1, b1, W2, b2, Wp, bp):
    B = user_id.shape[0]
    uid = user_id.astype(jnp.int32)
    iid = item_id.astype(jnp.int32)
    pos = jnp.arange(B, dtype=jnp.int32)
    su, ou = lax.sort_key_val(uid, pos)
    si, oi = lax.sort_key_val(iid, pos)
    V0 = P.shape[0]
    t0 = ((V0 - 2) // WIN) * WIN
    tpad = ((0, 0), (0, TAILW - (V0 - t0)))
    TP, TU, TQ, TV = (jnp.pad(T[:, t0:], tpad) for T in (P.T, U.T, Q.T, V.T))
    rows_u, rows_i = _sc_stream_gather(su, ou, si, oi, P.T, U.T, Q.T, V.T,
                                       TP, TU, TQ, TV)
    return _tc_head(rows_u, rows_i,
                    W1, b1[None, :], W2, b2[None, :], Wp, bp[None, :])


# speedup vs baseline: 3.2068x; 1.0376x over previous
import functools

import jax
import jax.numpy as jnp
from jax import lax
from jax.experimental import pallas as pl
from jax.experimental.pallas import tpu as pltpu
from jax.experimental.pallas import tpu_sc as plsc

WIN = 384
TAILW = 128


def _sc_stream_gather(su, ou, si, oi, PT, UT, QT, VT, TP, TU, TQ, TV):
    B = su.shape[0]
    Df = PT.shape[0]
    V = PT.shape[1]
    w_last = (V - 2) // WIN
    info = plsc.get_sparse_core_info()
    NC, NS = info.num_cores, info.num_subcores
    NW = NC * NS
    b_per_w = B // NW
    W2 = 2 * Df

    mesh = plsc.VectorSubcoreMesh(core_axis_name="c", subcore_axis_name="s")

    @functools.partial(
        pl.kernel,
        mesh=mesh,
        compiler_params=pltpu.CompilerParams(needs_layout_passes=False),
        out_type=[jax.ShapeDtypeStruct((B, W2), jnp.float32)] * 2,
        scratch_types=[
            pltpu.VMEM((b_per_w + 16,), jnp.int32),
            pltpu.VMEM((b_per_w,), jnp.int32),
            pltpu.VMEM((Df, WIN), jnp.float32),
            pltpu.VMEM((Df, WIN), jnp.float32),
            pltpu.VMEM((b_per_w, W2), jnp.float32),
            pltpu.SemaphoreType.DMA,
            pltpu.SemaphoreType.DMA,
        ],
    )
    def gather_kernel(su_hbm, ou_hbm, si_hbm, oi_hbm, pt, ut, qt, vt,
                      tp, tu, tq, tv,
                      out_u, out_i,
                      vs_vm, dest_v, blk_a, blk_b, rows, sa, sb):
        wid = lax.axis_index("s") * NC + lax.axis_index("c")
        base = wid * b_per_w

        def run_stream(v_hbm, o_hbm, ta, tb, tail_a, tail_b, out):
            pltpu.sync_copy(v_hbm.at[pl.ds(base, b_per_w)],
                            vs_vm.at[pl.ds(0, b_per_w)])

            def body(j, cur_win):
                v = jnp.minimum(vs_vm[pl.ds(j, 16)][0], V - 2)
                win = lax.div(v, WIN)
                changed = jnp.not_equal(win, cur_win)

                @pl.when(jnp.logical_and(changed, win < w_last))
                def _():
                    off = pl.multiple_of(win * WIN, WIN)
                    ca = pltpu.async_copy(ta.at[:, pl.ds(off, WIN)], blk_a, sa)
                    cb = pltpu.async_copy(tb.at[:, pl.ds(off, WIN)], blk_b, sb)
                    ca.wait()
                    cb.wait()

                @pl.when(jnp.logical_and(changed, win >= w_last))
                def _():
                    ca = pltpu.async_copy(tail_a,
                                          blk_a.at[:, pl.ds(0, TAILW)], sa)
                    cb = pltpu.async_copy(tail_b,
                                          blk_b.at[:, pl.ds(0, TAILW)], sb)
                    ca.wait()
                    cb.wait()

                c = jnp.full((16,), lax.rem(v, WIN), dtype=jnp.int32)
                for k in range(Df // 16):
                    r = jnp.arange(16 * k, 16 * (k + 1), dtype=jnp.int32)
                    rows[j, pl.ds(16 * k, 16)] = plsc.load_gather(blk_a, [r, c])
                    rows[j, pl.ds(Df + 16 * k, 16)] = plsc.load_gather(
                        blk_b, [r, c])
                return win

            lax.fori_loop(0, b_per_w, body, jnp.int32(-1))

            pltpu.sync_copy(o_hbm.at[pl.ds(base, b_per_w)], dest_v)
            for k in range(b_per_w // 128):
                pltpu.sync_copy(
                    rows.at[pl.ds(128 * k, 128)],
                    out.at[dest_v.at[pl.ds(128 * k, 128)]])

        run_stream(su_hbm, ou_hbm, pt, ut, tp, tu, out_u)
        run_stream(si_hbm, oi_hbm, qt, vt, tq, tv, out_i)

    return gather_kernel(su, ou, si, oi, PT, UT, QT, VT, TP, TU, TQ, TV)


def _head_body(pu_ref, qv_ref, w1_ref, b1_ref, w2_ref, b2_ref,
               wp_ref, bp_ref, out_ref):
    pu = pu_ref[...]
    qv = qv_ref[...]
    D = pu.shape[1] // 2
    gmf = pu[:, :D] * qv[:, :D]
    x = jnp.concatenate([pu[:, D:], qv[:, D:]], axis=1)
    h = lax.dot_general(x, w1_ref[...], (((1,), (1,)), ((), ())),
                        preferred_element_type=jnp.float32)
    h = jnp.maximum(h + b1_ref[...], 0.0)
    mlp = lax.dot_general(h, w2_ref[...], (((1,), (1,)), ((), ())),
                          preferred_element_type=jnp.float32)
    mlp = jnp.maximum(mlp + b2_ref[...], 0.0)
    con = jnp.concatenate([gmf, mlp], axis=1)
    z = jnp.sum(con * wp_ref[...], axis=1, keepdims=True)
    out_ref[...] = 1.0 / (1.0 + jnp.exp(-(z + bp_ref[0, 0])))


def _tc_head(rows_u, rows_i, W1, b1, W2, b2, Wp, bp, interpret=False):
    B, W = rows_u.shape
    BLK = 2048
    grid = (B // BLK,)
    blk_spec = pl.BlockSpec((BLK, W), lambda i: (i, 0))
    full = lambda shape: pl.BlockSpec(shape, lambda i: (0, 0))
    return pl.pallas_call(
        _head_body,
        grid=grid,
        in_specs=[
            blk_spec, blk_spec,
            full(W1.shape), full(b1.shape),
            full(W2.shape), full(b2.shape),
            full(Wp.shape),
            pl.BlockSpec(memory_space=pltpu.SMEM),
        ],
        out_specs=pl.BlockSpec((BLK, 1), lambda i: (i, 0)),
        out_shape=jax.ShapeDtypeStruct((B, 1), jnp.float32),
        compiler_params=pltpu.CompilerParams(
            dimension_semantics=("arbitrary",)),
        interpret=interpret,
    )(rows_u, rows_i, W1, b1, W2, b2, Wp, bp)


def kernel(user_id, item_id, P, Q, U, V, W1, b1, W2, b2, Wp, bp):
    B = user_id.shape[0]
    uid = user_id.astype(jnp.int32)
    iid = item_id.astype(jnp.int32)
    pos = jnp.arange(B, dtype=jnp.int32)
    su, ou = lax.sort_key_val(uid, pos)
    si, oi = lax.sort_key_val(iid, pos)
    V0 = P.shape[0]
    t0 = ((V0 - 2) // WIN) * WIN
    tpad = ((0, 0), (0, TAILW - (V0 - t0)))
    TP, TU, TQ, TV = (jnp.pad(T[:, t0:], tpad) for T in (P.T, U.T, Q.T, V.T))
    rows_u, rows_i = _sc_stream_gather(su, ou, si, oi, P.T, U.T, Q.T, V.T,
                                       TP, TU, TQ, TV)
    return _tc_head(rows_u, rows_i,
                    W1, b1[None, :], W2, b2[None, :], Wp, bp[None, :])


# speedup vs baseline: 3.3273x; 1.0376x over previous
import functools

import jax
import jax.numpy as jnp
from jax import lax
from jax.experimental import pallas as pl
from jax.experimental.pallas import tpu as pltpu
from jax.experimental.pallas import tpu_sc as plsc

WIN = 640
TAILW = 384
ROWCHUNK = 128


def _sc_stream_gather(su, ou, si, oi, PT, UT, QT, VT, TP, TU, TQ, TV):
    B = su.shape[0]
    Df = PT.shape[0]
    V = PT.shape[1]
    w_last = (V - 2) // WIN
    info = plsc.get_sparse_core_info()
    NC, NS = info.num_cores, info.num_subcores
    NW = NC * NS
    b_per_w = B // NW
    W2 = 2 * Df

    mesh = plsc.VectorSubcoreMesh(core_axis_name="c", subcore_axis_name="s")

    @functools.partial(
        pl.kernel,
        mesh=mesh,
        compiler_params=pltpu.CompilerParams(needs_layout_passes=False),
        out_type=[jax.ShapeDtypeStruct((B, W2), jnp.float32)] * 2,
        scratch_types=[
            pltpu.VMEM((b_per_w + 16,), jnp.int32),
            pltpu.VMEM((b_per_w,), jnp.int32),
            pltpu.VMEM((Df, WIN), jnp.float32),
            pltpu.VMEM((Df, WIN), jnp.float32),
            pltpu.VMEM((ROWCHUNK, W2), jnp.float32),
            pltpu.SemaphoreType.DMA,
            pltpu.SemaphoreType.DMA,
        ],
    )
    def gather_kernel(su_hbm, ou_hbm, si_hbm, oi_hbm, pt, ut, qt, vt,
                      tp, tu, tq, tv,
                      out_u, out_i,
                      vs_vm, dest_v, blk_a, blk_b, rows, sa, sb):
        wid = lax.axis_index("s") * NC + lax.axis_index("c")
        base = wid * b_per_w

        def run_stream(v_hbm, o_hbm, ta, tb, tail_a, tail_b, out):
            pltpu.sync_copy(v_hbm.at[pl.ds(base, b_per_w)],
                            vs_vm.at[pl.ds(0, b_per_w)])
            pltpu.sync_copy(o_hbm.at[pl.ds(base, b_per_w)], dest_v)

            def chunk(cidx, cur_win0):
                def body(jj, cur_win):
                    j = cidx * ROWCHUNK + jj
                    v = jnp.minimum(vs_vm[pl.ds(j, 16)][0], V - 2)
                    win = lax.div(v, WIN)
                    changed = jnp.not_equal(win, cur_win)

                    @pl.when(jnp.logical_and(changed, win < w_last))
                    def _():
                        off = pl.multiple_of(win * WIN, WIN)
                        ca = pltpu.async_copy(ta.at[:, pl.ds(off, WIN)],
                                              blk_a, sa)
                        cb = pltpu.async_copy(tb.at[:, pl.ds(off, WIN)],
                                              blk_b, sb)
                        ca.wait()
                        cb.wait()

                    @pl.when(jnp.logical_and(changed, win >= w_last))
                    def _():
                        ca = pltpu.async_copy(tail_a,
                                              blk_a.at[:, pl.ds(0, TAILW)], sa)
                        cb = pltpu.async_copy(tail_b,
                                              blk_b.at[:, pl.ds(0, TAILW)], sb)
                        ca.wait()
                        cb.wait()

                    c = jnp.full((16,), lax.rem(v, WIN), dtype=jnp.int32)
                    for k in range(Df // 16):
                        r = jnp.arange(16 * k, 16 * (k + 1), dtype=jnp.int32)
                        rows[jj, pl.ds(16 * k, 16)] = plsc.load_gather(
                            blk_a, [r, c])
                        rows[jj, pl.ds(Df + 16 * k, 16)] = plsc.load_gather(
                            blk_b, [r, c])
                    return win

                cur_win0 = lax.fori_loop(0, ROWCHUNK, body, cur_win0)
                pltpu.sync_copy(
                    rows,
                    out.at[dest_v.at[pl.ds(cidx * ROWCHUNK, ROWCHUNK)]])
                return cur_win0

            lax.fori_loop(0, b_per_w // ROWCHUNK, chunk, jnp.int32(-1))

        run_stream(su_hbm, ou_hbm, pt, ut, tp, tu, out_u)
        run_stream(si_hbm, oi_hbm, qt, vt, tq, tv, out_i)

    return gather_kernel(su, ou, si, oi, PT, UT, QT, VT, TP, TU, TQ, TV)


def _head_body(pu_ref, qv_ref, w1_ref, b1_ref, w2_ref, b2_ref,
               wp_ref, bp_ref, out_ref):
    pu = pu_ref[...]
    qv = qv_ref[...]
    D = pu.shape[1] // 2
    gmf = pu[:, :D] * qv[:, :D]
    x = jnp.concatenate([pu[:, D:], qv[:, D:]], axis=1)
    h = lax.dot_general(x, w1_ref[...], (((1,), (1,)), ((), ())),
                        preferred_element_type=jnp.float32)
    h = jnp.maximum(h + b1_ref[...], 0.0)
    mlp = lax.dot_general(h, w2_ref[...], (((1,), (1,)), ((), ())),
                          preferred_element_type=jnp.float32)
    mlp = jnp.maximum(mlp + b2_ref[...], 0.0)
    con = jnp.concatenate([gmf, mlp], axis=1)
    z = jnp.sum(con * wp_ref[...], axis=1, keepdims=True)
    out_ref[...] = 1.0 / (1.0 + jnp.exp(-(z + bp_ref[0, 0])))


def _tc_head(rows_u, rows_i, W1, b1, W2, b2, Wp, bp, interpret=False):
    B, W = rows_u.shape
    BLK = 2048
    grid = (B // BLK,)
    blk_spec = pl.BlockSpec((BLK, W), lambda i: (i, 0))
    full = lambda shape: pl.BlockSpec(shape, lambda i: (0, 0))
    return pl.pallas_call(
        _head_body,
        grid=grid,
        in_specs=[
            blk_spec, blk_spec,
            full(W1.shape), full(b1.shape),
            full(W2.shape), full(b2.shape),
            full(Wp.shape),
            pl.BlockSpec(memory_space=pltpu.SMEM),
        ],
        out_specs=pl.BlockSpec((BLK, 1), lambda i: (i, 0)),
        out_shape=jax.ShapeDtypeStruct((B, 1), jnp.float32),
        compiler_params=pltpu.CompilerParams(
            dimension_semantics=("arbitrary",)),
        interpret=interpret,
    )(rows_u, rows_i, W1, b1, W2, b2, Wp, bp)


def kernel(user_id, item_id, P, Q, U, V, W1, b1, W2, b2, Wp, bp):
    B = user_id.shape[0]
    uid = user_id.astype(jnp.int32)
    iid = item_id.astype(jnp.int32)
    pos = jnp.arange(B, dtype=jnp.int32)
    su, ou = lax.sort_key_val(uid, pos)
    si, oi = lax.sort_key_val(iid, pos)
    V0 = P.shape[0]
    t0 = ((V0 - 2) // WIN) * WIN
    tpad = ((0, 0), (0, TAILW - (V0 - t0)))
    TP, TU, TQ, TV = (jnp.pad(T[:, t0:], tpad) for T in (P.T, U.T, Q.T, V.T))
    rows_u, rows_i = _sc_stream_gather(su, ou, si, oi, P.T, U.T, Q.T, V.T,
                                       TP, TU, TQ, TV)
    return _tc_head(rows_u, rows_i,
                    W1, b1[None, :], W2, b2[None, :], Wp, bp[None, :])


# speedup vs baseline: 3.3429x; 1.0047x over previous
import functools

import jax
import jax.numpy as jnp
from jax import lax
from jax.experimental import pallas as pl
from jax.experimental.pallas import tpu as pltpu
from jax.experimental.pallas import tpu_sc as plsc

WIN = 768
TAILW = 384
ROWCHUNK = 128


def _sc_stream_gather(su, ou, si, oi, PT, UT, QT, VT, TP, TU, TQ, TV):
    B = su.shape[0]
    Df = PT.shape[0]
    V = PT.shape[1]
    w_last = (V - 2) // WIN
    info = plsc.get_sparse_core_info()
    NC, NS = info.num_cores, info.num_subcores
    NW = NC * NS
    b_per_w = B // NW
    W2 = 2 * Df

    mesh = plsc.VectorSubcoreMesh(core_axis_name="c", subcore_axis_name="s")

    @functools.partial(
        pl.kernel,
        mesh=mesh,
        compiler_params=pltpu.CompilerParams(needs_layout_passes=False),
        out_type=[jax.ShapeDtypeStruct((B, W2), jnp.float32)] * 2,
        scratch_types=[
            pltpu.VMEM((b_per_w + 16,), jnp.int32),
            pltpu.VMEM((b_per_w,), jnp.int32),
            pltpu.VMEM((Df, WIN), jnp.float32),
            pltpu.VMEM((Df, WIN), jnp.float32),
            pltpu.VMEM((ROWCHUNK, W2), jnp.float32),
            pltpu.SemaphoreType.DMA,
            pltpu.SemaphoreType.DMA,
        ],
    )
    def gather_kernel(su_hbm, ou_hbm, si_hbm, oi_hbm, pt, ut, qt, vt,
                      tp, tu, tq, tv,
                      out_u, out_i,
                      vs_vm, dest_v, blk_a, blk_b, rows, sa, sb):
        wid = lax.axis_index("s") * NC + lax.axis_index("c")
        base = wid * b_per_w

        def run_stream(v_hbm, o_hbm, ta, tb, tail_a, tail_b, out):
            pltpu.sync_copy(v_hbm.at[pl.ds(base, b_per_w)],
                            vs_vm.at[pl.ds(0, b_per_w)])
            pltpu.sync_copy(o_hbm.at[pl.ds(base, b_per_w)], dest_v)

            def chunk(cidx, cur_win0):
                def body(jj, cur_win):
                    j = cidx * ROWCHUNK + jj
                    v = jnp.minimum(vs_vm[pl.ds(j, 16)][0], V - 2)
                    win = lax.div(v, WIN)
                    changed = jnp.not_equal(win, cur_win)

                    @pl.when(jnp.logical_and(changed, win < w_last))
                    def _():
                        off = pl.multiple_of(win * WIN, WIN)
                        ca = pltpu.async_copy(ta.at[:, pl.ds(off, WIN)],
                                              blk_a, sa)
                        cb = pltpu.async_copy(tb.at[:, pl.ds(off, WIN)],
                                              blk_b, sb)
                        ca.wait()
                        cb.wait()

                    @pl.when(jnp.logical_and(changed, win >= w_last))
                    def _():
                        ca = pltpu.async_copy(tail_a,
                                              blk_a.at[:, pl.ds(0, TAILW)], sa)
                        cb = pltpu.async_copy(tail_b,
                                              blk_b.at[:, pl.ds(0, TAILW)], sb)
                        ca.wait()
                        cb.wait()

                    c = jnp.full((16,), lax.rem(v, WIN), dtype=jnp.int32)
                    for k in range(Df // 16):
                        r = jnp.arange(16 * k, 16 * (k + 1), dtype=jnp.int32)
                        rows[jj, pl.ds(16 * k, 16)] = plsc.load_gather(
                            blk_a, [r, c])
                        rows[jj, pl.ds(Df + 16 * k, 16)] = plsc.load_gather(
                            blk_b, [r, c])
                    return win

                cur_win0 = lax.fori_loop(0, ROWCHUNK, body, cur_win0)
                pltpu.sync_copy(
                    rows,
                    out.at[dest_v.at[pl.ds(cidx * ROWCHUNK, ROWCHUNK)]])
                return cur_win0

            lax.fori_loop(0, b_per_w // ROWCHUNK, chunk, jnp.int32(-1))

        run_stream(su_hbm, ou_hbm, pt, ut, tp, tu, out_u)
        run_stream(si_hbm, oi_hbm, qt, vt, tq, tv, out_i)

    return gather_kernel(su, ou, si, oi, PT, UT, QT, VT, TP, TU, TQ, TV)


def _head_body(pu_ref, qv_ref, w1_ref, b1_ref, w2_ref, b2_ref,
               wp_ref, bp_ref, out_ref):
    pu = pu_ref[...]
    qv = qv_ref[...]
    D = pu.shape[1] // 2
    gmf = pu[:, :D] * qv[:, :D]
    x = jnp.concatenate([pu[:, D:], qv[:, D:]], axis=1)
    h = lax.dot_general(x, w1_ref[...], (((1,), (1,)), ((), ())),
                        preferred_element_type=jnp.float32)
    h = jnp.maximum(h + b1_ref[...], 0.0)
    mlp = lax.dot_general(h, w2_ref[...], (((1,), (1,)), ((), ())),
                          preferred_element_type=jnp.float32)
    mlp = jnp.maximum(mlp + b2_ref[...], 0.0)
    con = jnp.concatenate([gmf, mlp], axis=1)
    z = jnp.sum(con * wp_ref[...], axis=1, keepdims=True)
    out_ref[...] = 1.0 / (1.0 + jnp.exp(-(z + bp_ref[0, 0])))


def _tc_head(rows_u, rows_i, W1, b1, W2, b2, Wp, bp, interpret=False):
    B, W = rows_u.shape
    BLK = 2048
    grid = (B // BLK,)
    blk_spec = pl.BlockSpec((BLK, W), lambda i: (i, 0))
    full = lambda shape: pl.BlockSpec(shape, lambda i: (0, 0))
    return pl.pallas_call(
        _head_body,
        grid=grid,
        in_specs=[
            blk_spec, blk_spec,
            full(W1.shape), full(b1.shape),
            full(W2.shape), full(b2.shape),
            full(Wp.shape),
            pl.BlockSpec(memory_space=pltpu.SMEM),
        ],
        out_specs=pl.BlockSpec((BLK, 1), lambda i: (i, 0)),
        out_shape=jax.ShapeDtypeStruct((B, 1), jnp.float32),
        compiler_params=pltpu.CompilerParams(
            dimension_semantics=("arbitrary",)),
        interpret=interpret,
    )(rows_u, rows_i, W1, b1, W2, b2, Wp, bp)


def kernel(user_id, item_id, P, Q, U, V, W1, b1, W2, b2, Wp, bp):
    B = user_id.shape[0]
    uid = user_id.astype(jnp.int32)
    iid = item_id.astype(jnp.int32)
    pos = jnp.arange(B, dtype=jnp.int32)
    su, ou = lax.sort_key_val(uid, pos)
    si, oi = lax.sort_key_val(iid, pos)
    V0 = P.shape[0]
    t0 = ((V0 - 2) // WIN) * WIN
    tpad = ((0, 0), (0, TAILW - (V0 - t0)))
    TP, TU, TQ, TV = (jnp.pad(T[:, t0:], tpad) for T in (P.T, U.T, Q.T, V.T))
    rows_u, rows_i = _sc_stream_gather(su, ou, si, oi, P.T, U.T, Q.T, V.T,
                                       TP, TU, TQ, TV)
    return _tc_head(rows_u, rows_i,
                    W1, b1[None, :], W2, b2[None, :], Wp, bp[None, :])
